# baseline (speedup 1.0000x reference)
import functools

import jax
import jax.numpy as jnp
from jax import lax
from jax.experimental import pallas as pl
from jax.experimental.pallas import tpu as pltpu
from jax.experimental.pallas import tpu_sc as plsc

NC, NS, L = 2, 16, 16
NW = NC * NS
K_CHUNKS = 4


def _logits_block(x_ref, w_ref, b_ref, lt_ref):
    lt = lax.dot_general(w_ref[...], x_ref[...], (((1,), (1,)), ((), ())),
                         preferred_element_type=jnp.float32)
    lt_ref[...] = (lt + b_ref[...])[None]


def _tc_logits(xc, W, b, blk):
    tokens, C = xc.shape
    E = W.shape[0]
    return pl.pallas_call(
        _logits_block,
        grid=(tokens // blk,),
        in_specs=[
            pl.BlockSpec((blk, C), lambda i: (i, 0)),
            pl.BlockSpec((E, C), lambda i: (0, 0)),
            pl.BlockSpec((E, 1), lambda i: (0, 0)),
        ],
        out_specs=pl.BlockSpec((1, E, blk), lambda i: (i, 0, 0)),
        out_shape=jax.ShapeDtypeStruct((tokens // blk, E, blk), jnp.float32),
    )(xc, W, b.reshape(E, 1))


def _sc_route(lt3, num_experts):
    nw, E, chunk = lt3.shape
    mesh = plsc.VectorSubcoreMesh(core_axis_name="c", subcore_axis_name="s")
    shape = jax.ShapeDtypeStruct(lt3.shape, jnp.float32)

    @functools.partial(
        pl.kernel, mesh=mesh,
        out_type=[shape, shape],
        scratch_types=[
            pltpu.VMEM((E, chunk), jnp.float32),
            pltpu.VMEM((E, chunk), jnp.float32),
            pltpu.VMEM((E, chunk), jnp.float32),
        ],
    )
    def sc_kernel(lt_hbm, r_hbm, m_hbm, in_v, r_v, m_v):
        wid = lax.axis_index("s") * NC + lax.axis_index("c")
        pltpu.sync_copy(lt_hbm.at[wid], in_v)

        @pl.loop(0, chunk, step=2 * L)
        def _(c):
            for g in range(2):
                sl = pl.ds(c + g * L, L)
                m1 = in_v[0, sl]
                i1 = jnp.zeros((L,), jnp.int32)
                m2 = jnp.full((L,), -jnp.inf, jnp.float32)
                i2 = jnp.zeros((L,), jnp.int32)
                for e in range(1, num_experts):
                    v = in_v[e, sl]
                    e_vec = jnp.full((L,), e, jnp.int32)
                    i2n = jnp.where(v > m2, e_vec, i2)
                    m2n = jnp.where(v > m2, v, m2)
                    i2 = jnp.where(v > m1, i1, i2n)
                    m2 = jnp.where(v > m1, m1, m2n)
                    i1 = jnp.where(v > m1, e_vec, i1)
                    m1 = jnp.where(v > m1, v, m1)
                z0 = jnp.zeros((L,), jnp.float32)
                z1 = jnp.zeros((L,), jnp.float32)
                for e in range(0, num_experts, 2):
                    z0 = z0 + jnp.exp(in_v[e, sl] - m1)
                    z1 = z1 + jnp.exp(in_v[e + 1, sl] - m1)
                z = z0 + z1
                a = 1.0 / (1.0 + jnp.exp((jnp.exp(m2 - m1) - 1.0) / z))
                ones = jnp.ones((L,), jnp.float32)
                zeros = jnp.zeros((L,), jnp.float32)
                for e in range(num_experts):
                    sel1 = i1 == e
                    sel2 = i2 == e
                    m_v[e, sl] = jnp.where(sel1, ones, jnp.where(sel2, ones, zeros))
                    r_v[e, sl] = jnp.where(sel1, a, jnp.where(sel2, 1.0 - a, zeros))

        pltpu.sync_copy(r_v, r_hbm.at[wid])
        pltpu.sync_copy(m_v, m_hbm.at[wid])

    return sc_kernel(lt3)


def kernel(x, W, b):
    B, T, C = x.shape
    E = W.shape[0]
    tokens = B * T
    ctoks = tokens // K_CHUNKS
    chunk = ctoks // NW
    xf = x.reshape(tokens, C)
    outs = []
    for k in range(K_CHUNKS):
        xc = lax.slice_in_dim(xf, k * ctoks, (k + 1) * ctoks, axis=0)
        lt3 = _tc_logits(xc, W, b, chunk)
        outs.append(_sc_route(lt3, E))
    r3 = jnp.concatenate([o[0] for o in outs], axis=0)
    m3 = jnp.concatenate([o[1] for o in outs], axis=0)
    out = jnp.transpose(r3, (0, 2, 1)).reshape(B, T, E)
    mask = jnp.transpose(m3, (0, 2, 1)).reshape(B, T, E)
    return out, mask


# speedup vs baseline: 4.6853x; 4.6853x over previous
import functools

import jax
import jax.numpy as jnp
from jax.experimental import pallas as pl

INTERPRET = False


def _router_block_t(x_ref, w_ref, b_ref, out_ref, mask_ref, *, num_experts):
    logits = jnp.dot(x_ref[...], w_ref[...], preferred_element_type=jnp.float32)
    logits = logits + b_ref[...]
    lt = logits.T
    idx = jax.lax.broadcasted_iota(jnp.int32, lt.shape, 0)
    m1 = jnp.max(lt, axis=0, keepdims=True)
    i1 = jnp.min(jnp.where(lt == m1, idx, num_experts), axis=0, keepdims=True)
    l2 = jnp.where(idx == i1, -jnp.inf, lt)
    m2 = jnp.max(l2, axis=0, keepdims=True)
    i2 = jnp.min(jnp.where(l2 == m2, idx, num_experts), axis=0, keepdims=True)
    z = jnp.sum(jnp.exp(lt - m1), axis=0, keepdims=True)
    a = 1.0 / (1.0 + jnp.exp((jnp.exp(m2 - m1) - 1.0) / z))
    sel1 = idx == i1
    sel2 = idx == i2
    r_t = jnp.where(sel1, a, jnp.where(sel2, 1.0 - a, 0.0))
    mask_t = jnp.logical_or(sel1, sel2).astype(jnp.float32)
    out_ref[...] = r_t.T
    mask_ref[...] = mask_t.T


def kernel_a(x, W, b, blk=1024):
    B, T, C = x.shape
    E = W.shape[0]
    tokens = B * T
    xf = x.reshape(tokens, C)
    out, mask = pl.pallas_call(
        functools.partial(_router_block_t, num_experts=E),
        grid=(tokens // blk,),
        in_specs=[
            pl.BlockSpec((blk, C), lambda i: (i, 0)),
            pl.BlockSpec((C, E), lambda i: (0, 0)),
            pl.BlockSpec((1, E), lambda i: (0, 0)),
        ],
        out_specs=[
            pl.BlockSpec((blk, E), lambda i: (i, 0)),
            pl.BlockSpec((blk, E), lambda i: (i, 0)),
        ],
        out_shape=[
            jax.ShapeDtypeStruct((tokens, E), jnp.float32),
            jax.ShapeDtypeStruct((tokens, E), jnp.float32),
        ],
        interpret=INTERPRET,
    )(xf, W.T, b.reshape(1, E))
    return out.reshape(B, T, E), mask.reshape(B, T, E)


def _router_block_nt(x_ref, w_ref, b_ref, out_ref, mask_ref, *, num_experts):
    lt = jax.lax.dot_general(
        w_ref[...], x_ref[...], (((1,), (1,)), ((), ())),
        preferred_element_type=jnp.float32)
    lt = lt + b_ref[...]
    idx = jax.lax.broadcasted_iota(jnp.int32, lt.shape, 0)
    m1 = jnp.max(lt, axis=0, keepdims=True)
    i1 = jnp.min(jnp.where(lt == m1, idx, num_experts), axis=0, keepdims=True)
    l2 = jnp.where(idx == i1, -jnp.inf, lt)
    m2 = jnp.max(l2, axis=0, keepdims=True)
    i2 = jnp.min(jnp.where(l2 == m2, idx, num_experts), axis=0, keepdims=True)
    z = jnp.sum(jnp.exp(lt - m1), axis=0, keepdims=True)
    a = 1.0 / (1.0 + jnp.exp((jnp.exp(m2 - m1) - 1.0) / z))
    sel1 = idx == i1
    sel2 = idx == i2
    out_ref[...] = jnp.where(sel1, a, jnp.where(sel2, 1.0 - a, 0.0))
    mask_ref[...] = jnp.logical_or(sel1, sel2).astype(jnp.float32)


def kernel(x, W, b, blk=1024):
    B, T, C = x.shape
    E = W.shape[0]
    tokens = B * T
    xf = x.reshape(tokens, C)
    out_t, mask_t = pl.pallas_call(
        functools.partial(_router_block_nt, num_experts=E),
        grid=(tokens // blk,),
        in_specs=[
            pl.BlockSpec((blk, C), lambda i: (i, 0)),
            pl.BlockSpec((E, C), lambda i: (0, 0)),
            pl.BlockSpec((E, 1), lambda i: (0, 0)),
        ],
        out_specs=[
            pl.BlockSpec((E, blk), lambda i: (0, i)),
            pl.BlockSpec((E, blk), lambda i: (0, i)),
        ],
        out_shape=[
            jax.ShapeDtypeStruct((E, tokens), jnp.float32),
            jax.ShapeDtypeStruct((E, tokens), jnp.float32),
        ],
        interpret=INTERPRET,
    )(xf, W, b.reshape(E, 1))
    return (out_t.T.reshape(B, T, E), mask_t.T.reshape(B, T, E))
